# baseline (speedup 1.0000x reference)
import jax
import jax.numpy as jnp
from jax import lax
from jax.experimental import pallas as pl
from jax.experimental.pallas import tpu as pltpu
from jax.experimental.pallas import tpu_sc as plsc

H = 1024
S = 512
U = 64
N = S * U
R = 4096
NBLK = N // R
TPB = 8
NQ_PAD = 1024
STD_THETA = 1.0


_NC = 2
_NS = 16
_NW = _NC * _NS
_CHUNK = N // _NW
_LANES = 16


def _sc_gather_body(q_hbm, dtab_hbm, ktab_hbm, dout_hbm, kout_hbm,
                    idx_v, dtab_v, ktab_v, dout_v, kout_v):
    wid = lax.axis_index("s") * _NC + lax.axis_index("c")
    base = wid * _CHUNK
    pltpu.sync_copy(q_hbm.at[pl.ds(base, _CHUNK)], idx_v)
    pltpu.sync_copy(dtab_hbm, dtab_v)
    pltpu.sync_copy(ktab_hbm, ktab_v)
    for j in range(_CHUNK // _LANES):
        idx = idx_v[pl.ds(j * _LANES, _LANES)]
        dout_v[pl.ds(j * _LANES, _LANES)] = plsc.load_gather(dtab_v, [idx])
        kout_v[pl.ds(j * _LANES, _LANES)] = plsc.load_gather(ktab_v, [idx])
    pltpu.sync_copy(dout_v, dout_hbm.at[pl.ds(base, _CHUNK)])
    pltpu.sync_copy(kout_v, kout_hbm.at[pl.ds(base, _CHUNK)])


def _sc_gather(q_flat, dtab_pad, ktab_pad):
    mesh = plsc.VectorSubcoreMesh(core_axis_name="c", subcore_axis_name="s")
    f32 = jnp.float32
    call = pl.kernel(
        _sc_gather_body,
        mesh=mesh,
        compiler_params=pltpu.CompilerParams(needs_layout_passes=False),
        out_type=[jax.ShapeDtypeStruct((N,), f32),
                  jax.ShapeDtypeStruct((N,), f32)],
        scratch_types=[
            pltpu.VMEM((_CHUNK,), jnp.int32),
            pltpu.VMEM((NQ_PAD,), f32),
            pltpu.VMEM((NQ_PAD,), f32),
            pltpu.VMEM((_CHUNK,), f32),
            pltpu.VMEM((_CHUNK,), f32),
        ],
    )
    return call(q_flat, dtab_pad, ktab_pad)


_SQRT_HALF = 0.7071067811865476


def _gelu(x):
    return 0.5 * x * (1.0 + lax.erf(x * _SQRT_HALF))


def _mlp_body(x8_ref, w1t_ref, b1_ref, w2t_ref, b2_ref, w3t_ref, b3_ref,
              mu_ref, r2_ref):
    x = x8_ref[0]
    h = jnp.dot(w1t_ref[...], x, preferred_element_type=jnp.float32)
    h = _gelu(h + b1_ref[...])
    h = jnp.dot(w2t_ref[...], h, preferred_element_type=jnp.float32)
    h = _gelu(h + b2_ref[...])
    o = jnp.dot(w3t_ref[...], h, preferred_element_type=jnp.float32)
    o = _gelu(o + b3_ref[...])
    mu = o[0:1, :]
    logvar = o[1:2, :]
    std = jnp.maximum(jnp.exp(0.5 * logvar), 1e-8)
    r2 = (STD_THETA / std) ** 2
    mu_ref[0] = mu
    r2_ref[0] = r2


def _mlp_call(x8, w1t8, b1c, w2t, b2c, w3t8, b3c):
    f32 = jnp.float32
    return pl.pallas_call(
        _mlp_body,
        grid=(NBLK,),
        in_specs=[
            pl.BlockSpec((1, 8, R), lambda i: (i, 0, 0)),
            pl.BlockSpec((H, 8), lambda i: (0, 0)),
            pl.BlockSpec((H, 1), lambda i: (0, 0)),
            pl.BlockSpec((H, H), lambda i: (0, 0)),
            pl.BlockSpec((H, 1), lambda i: (0, 0)),
            pl.BlockSpec((8, H), lambda i: (0, 0)),
            pl.BlockSpec((8, 1), lambda i: (0, 0)),
        ],
        out_specs=[
            pl.BlockSpec((1, 1, R), lambda i: (i, 0, 0)),
            pl.BlockSpec((1, 1, R), lambda i: (i, 0, 0)),
        ],
        out_shape=[jax.ShapeDtypeStruct((NBLK, 1, R), f32),
                   jax.ShapeDtypeStruct((NBLK, 1, R), f32)],
    )(x8, w1t8, b1c, w2t, b2c, w3t8, b3c)


def _scan_body(mu_ref, r2_ref, diff_ref, disc_ref, logits_ref, last_ref,
               b_scr, c_scr):
    ones = jnp.ones((1, U), jnp.float32)
    zeros = jnp.zeros((1, U), jnp.float32)
    NT = S // TPB

    def bwd(t, carry):
        b_prev, c_prev = carry
        row0 = (NT - 1 - t) * TPB
        r2t = r2_ref[pl.ds(row0, TPB), :]
        mut = mu_ref[pl.ds(row0, TPB), :]
        bs, cs = [None] * TPB, [None] * TPB
        for j in range(TPB - 1, -1, -1):
            r2j = r2t[j:j + 1, :]
            b_prev = 1.0 / (2.0 + r2j - b_prev)
            c_prev = b_prev * (c_prev + r2j * mut[j:j + 1, :])
            bs[j] = b_prev
            cs[j] = c_prev
        b_scr[pl.ds(row0, TPB), :] = jnp.concatenate(bs, axis=0)
        c_scr[pl.ds(row0, TPB), :] = jnp.concatenate(cs, axis=0)
        return (b_prev, c_prev)

    lax.fori_loop(0, NT, bwd, (ones, zeros))

    def fwd(t, abil):
        row0 = t * TPB
        bt = b_scr[pl.ds(row0, TPB), :]
        ct = c_scr[pl.ds(row0, TPB), :]
        dt = diff_ref[pl.ds(row0, TPB), :]
        kt = disc_ref[pl.ds(row0, TPB), :]
        ls = [None] * TPB
        for j in range(TPB):
            abil = bt[j:j + 1, :] * abil + ct[j:j + 1, :]
            ls[j] = kt[j:j + 1, :] * (abil - dt[j:j + 1, :])
        logits_ref[pl.ds(row0, TPB), :] = jnp.concatenate(ls, axis=0)
        return abil

    a_last = lax.fori_loop(0, NT, fwd, zeros)
    last_ref[...] = a_last


def _scan_call(mu_t, r2_t, diff_t, disc_t):
    f32 = jnp.float32
    return pl.pallas_call(
        _scan_body,
        out_shape=[jax.ShapeDtypeStruct((S, U), f32),
                   jax.ShapeDtypeStruct((1, U), f32)],
        scratch_shapes=[pltpu.VMEM((S, U), f32), pltpu.VMEM((S, U), f32)],
    )(mu_t, r2_t, diff_t, disc_t)


def kernel(mask, q_id, kmap, resp, diff_mu_w, disc_mu_w, W1, b1, W2, b2, W3, b3):
    f32 = jnp.float32
    q_flat = q_id.T.reshape(N).astype(jnp.int32)
    resp_flat = resp.T.reshape(N).astype(f32)

    dtab_pad = jnp.zeros((NQ_PAD,), f32).at[:diff_mu_w.shape[0]].set(diff_mu_w[:, 0])
    ktab_pad = jnp.zeros((NQ_PAD,), f32).at[:disc_mu_w.shape[0]].set(disc_mu_w[:, 0])

    diff_flat, disc_flat = _sc_gather(q_flat, dtab_pad, ktab_pad)

    x = jnp.stack([diff_flat, disc_flat, resp_flat], axis=0)
    x8 = jnp.zeros((8, N), f32).at[:3].set(x)
    x8 = x8.reshape(8, NBLK, R).transpose(1, 0, 2)

    w1t8 = jnp.zeros((H, 8), f32).at[:, :3].set(W1.T)
    w3t8 = jnp.zeros((8, H), f32).at[:2].set(W3.T)
    b3c = jnp.zeros((8, 1), f32).at[:2, 0].set(b3)

    mu3, r23 = _mlp_call(x8, w1t8, b1.reshape(H, 1), W2.T,
                         b2.reshape(H, 1), w3t8, b3c)

    mu_t = mu3.reshape(N).reshape(S, U)
    r2_t = r23.reshape(N).reshape(S, U)
    diff_t = diff_flat.reshape(S, U)
    disc_t = disc_flat.reshape(S, U)

    logits_t, last = _scan_call(mu_t, r2_t, diff_t, disc_t)

    return logits_t.T, last.reshape(U, 1)


# speedup vs baseline: 1.0087x; 1.0087x over previous
import jax
import jax.numpy as jnp
from jax import lax
from jax.experimental import pallas as pl
from jax.experimental.pallas import tpu as pltpu
from jax.experimental.pallas import tpu_sc as plsc

H = 1024
S = 512
U = 64
N = S * U
R = 4096
NBLK = N // R
TPB = 8
NQ_PAD = 1024
STD_THETA = 1.0


_NC = 2
_NS = 16
_NW = _NC * _NS
_CHUNK = N // _NW
_LANES = 16


def _sc_gather_body(q_hbm, dtab_hbm, ktab_hbm, dout_hbm, kout_hbm,
                    idx_v, dtab_v, ktab_v, dout_v, kout_v):
    wid = lax.axis_index("s") * _NC + lax.axis_index("c")
    base = wid * _CHUNK
    pltpu.sync_copy(q_hbm.at[pl.ds(base, _CHUNK)], idx_v)
    pltpu.sync_copy(dtab_hbm, dtab_v)
    pltpu.sync_copy(ktab_hbm, ktab_v)
    for j in range(_CHUNK // _LANES):
        idx = idx_v[pl.ds(j * _LANES, _LANES)]
        dout_v[pl.ds(j * _LANES, _LANES)] = plsc.load_gather(dtab_v, [idx])
        kout_v[pl.ds(j * _LANES, _LANES)] = plsc.load_gather(ktab_v, [idx])
    pltpu.sync_copy(dout_v, dout_hbm.at[pl.ds(base, _CHUNK)])
    pltpu.sync_copy(kout_v, kout_hbm.at[pl.ds(base, _CHUNK)])


def _sc_gather(q_flat, dtab_pad, ktab_pad):
    mesh = plsc.VectorSubcoreMesh(core_axis_name="c", subcore_axis_name="s")
    f32 = jnp.float32
    call = pl.kernel(
        _sc_gather_body,
        mesh=mesh,
        compiler_params=pltpu.CompilerParams(needs_layout_passes=False),
        out_type=[jax.ShapeDtypeStruct((N,), f32),
                  jax.ShapeDtypeStruct((N,), f32)],
        scratch_types=[
            pltpu.VMEM((_CHUNK,), jnp.int32),
            pltpu.VMEM((NQ_PAD,), f32),
            pltpu.VMEM((NQ_PAD,), f32),
            pltpu.VMEM((_CHUNK,), f32),
            pltpu.VMEM((_CHUNK,), f32),
        ],
    )
    return call(q_flat, dtab_pad, ktab_pad)


_SQRT_HALF = 0.7071067811865476


def _gelu(x):
    return 0.5 * x * (1.0 + lax.erf(x * _SQRT_HALF))


def _mlp_body(x8_ref, w1t_ref, b1_ref, w2t_ref, b2_ref, w3t_ref, b3_ref,
              mu_ref, r2_ref):
    x = x8_ref[0]
    h = jnp.dot(w1t_ref[...], x, preferred_element_type=jnp.float32)
    h = _gelu(h + b1_ref[...])
    h = jnp.dot(w2t_ref[...], h.astype(jnp.bfloat16),
                preferred_element_type=jnp.float32)
    h = _gelu(h + b2_ref[...])
    o = jnp.dot(w3t_ref[...], h, preferred_element_type=jnp.float32)
    o = _gelu(o + b3_ref[...])
    mu = o[0:1, :]
    logvar = o[1:2, :]
    std = jnp.maximum(jnp.exp(0.5 * logvar), 1e-8)
    r2 = (STD_THETA / std) ** 2
    mu_ref[0] = mu
    r2_ref[0] = r2


def _mlp_call(x8, w1t8, b1c, w2t, b2c, w3t8, b3c):
    f32 = jnp.float32
    return pl.pallas_call(
        _mlp_body,
        grid=(NBLK,),
        in_specs=[
            pl.BlockSpec((1, 8, R), lambda i: (i, 0, 0)),
            pl.BlockSpec((H, 8), lambda i: (0, 0)),
            pl.BlockSpec((H, 1), lambda i: (0, 0)),
            pl.BlockSpec((H, H), lambda i: (0, 0)),
            pl.BlockSpec((H, 1), lambda i: (0, 0)),
            pl.BlockSpec((8, H), lambda i: (0, 0)),
            pl.BlockSpec((8, 1), lambda i: (0, 0)),
        ],
        out_specs=[
            pl.BlockSpec((1, 1, R), lambda i: (i, 0, 0)),
            pl.BlockSpec((1, 1, R), lambda i: (i, 0, 0)),
        ],
        out_shape=[jax.ShapeDtypeStruct((NBLK, 1, R), f32),
                   jax.ShapeDtypeStruct((NBLK, 1, R), f32)],
    )(x8, w1t8, b1c, w2t, b2c, w3t8, b3c)


def _scan_body(mu_ref, r2_ref, diff_ref, disc_ref, logits_ref, last_ref,
               b_scr, c_scr):
    ones = jnp.ones((1, U), jnp.float32)
    zeros = jnp.zeros((1, U), jnp.float32)
    NT = S // TPB

    def bwd(t, carry):
        b_prev, c_prev = carry
        row0 = (NT - 1 - t) * TPB
        r2t = r2_ref[pl.ds(row0, TPB), :]
        mut = mu_ref[pl.ds(row0, TPB), :]
        bs, cs = [None] * TPB, [None] * TPB
        for j in range(TPB - 1, -1, -1):
            r2j = r2t[j:j + 1, :]
            b_prev = 1.0 / (2.0 + r2j - b_prev)
            c_prev = b_prev * (c_prev + r2j * mut[j:j + 1, :])
            bs[j] = b_prev
            cs[j] = c_prev
        b_scr[pl.ds(row0, TPB), :] = jnp.concatenate(bs, axis=0)
        c_scr[pl.ds(row0, TPB), :] = jnp.concatenate(cs, axis=0)
        return (b_prev, c_prev)

    lax.fori_loop(0, NT, bwd, (ones, zeros))

    def fwd(t, abil):
        row0 = t * TPB
        bt = b_scr[pl.ds(row0, TPB), :]
        ct = c_scr[pl.ds(row0, TPB), :]
        dt = diff_ref[pl.ds(row0, TPB), :]
        kt = disc_ref[pl.ds(row0, TPB), :]
        ls = [None] * TPB
        for j in range(TPB):
            abil = bt[j:j + 1, :] * abil + ct[j:j + 1, :]
            ls[j] = kt[j:j + 1, :] * (abil - dt[j:j + 1, :])
        logits_ref[pl.ds(row0, TPB), :] = jnp.concatenate(ls, axis=0)
        return abil

    a_last = lax.fori_loop(0, NT, fwd, zeros)
    last_ref[...] = a_last


def _scan_call(mu_t, r2_t, diff_t, disc_t):
    f32 = jnp.float32
    return pl.pallas_call(
        _scan_body,
        out_shape=[jax.ShapeDtypeStruct((S, U), f32),
                   jax.ShapeDtypeStruct((1, U), f32)],
        scratch_shapes=[pltpu.VMEM((S, U), f32), pltpu.VMEM((S, U), f32)],
    )(mu_t, r2_t, diff_t, disc_t)


def kernel(mask, q_id, kmap, resp, diff_mu_w, disc_mu_w, W1, b1, W2, b2, W3, b3):
    f32 = jnp.float32
    q_flat = q_id.T.reshape(N).astype(jnp.int32)
    resp_flat = resp.T.reshape(N).astype(f32)

    dtab_pad = jnp.zeros((NQ_PAD,), f32).at[:diff_mu_w.shape[0]].set(diff_mu_w[:, 0])
    ktab_pad = jnp.zeros((NQ_PAD,), f32).at[:disc_mu_w.shape[0]].set(disc_mu_w[:, 0])

    diff_flat, disc_flat = _sc_gather(q_flat, dtab_pad, ktab_pad)

    x = jnp.stack([diff_flat, disc_flat, resp_flat], axis=0)
    x8 = jnp.zeros((8, N), f32).at[:3].set(x)
    x8 = x8.reshape(8, NBLK, R).transpose(1, 0, 2)

    w1t8 = jnp.zeros((H, 8), f32).at[:, :3].set(W1.T)
    w3t8 = jnp.zeros((8, H), f32).at[:2].set(W3.T)
    b3c = jnp.zeros((8, 1), f32).at[:2, 0].set(b3)

    mu3, r23 = _mlp_call(x8, w1t8, b1.reshape(H, 1),
                         W2.T.astype(jnp.bfloat16),
                         b2.reshape(H, 1), w3t8, b3c)

    mu_t = mu3.reshape(N).reshape(S, U)
    r2_t = r23.reshape(N).reshape(S, U)
    diff_t = diff_flat.reshape(S, U)
    disc_t = disc_flat.reshape(S, U)

    logits_t, last = _scan_call(mu_t, r2_t, diff_t, disc_t)

    return logits_t.T, last.reshape(U, 1)


# speedup vs baseline: 2.2906x; 2.2710x over previous
import jax
import jax.numpy as jnp
from jax import lax
from jax.experimental import pallas as pl
from jax.experimental.pallas import tpu as pltpu
from jax.experimental.pallas import tpu_sc as plsc

H = 1024
S = 512
U = 64
N = S * U
NQ_PAD = 1024
NC_PAD = 2048
TPB = 8
STD_THETA = 1.0


_SQRT_HALF = 0.7071067811865476


def _gelu(x):
    return 0.5 * x * (1.0 + lax.erf(x * _SQRT_HALF))


def _mlp_body(x8_ref, w1t_ref, b1_ref, w2t_ref, b2_ref, w3t_ref, b3_ref,
              mu_ref, r2_ref):
    x = x8_ref[...]
    h = jnp.dot(w1t_ref[...], x, preferred_element_type=jnp.float32)
    h = _gelu(h + b1_ref[...])
    h = jnp.dot(w2t_ref[...], h.astype(jnp.bfloat16),
                preferred_element_type=jnp.float32)
    h = _gelu(h + b2_ref[...])
    o = jnp.dot(w3t_ref[...], h, preferred_element_type=jnp.float32)
    o = _gelu(o + b3_ref[...])
    mu = o[0:1, :]
    logvar = o[1:2, :]
    std = jnp.maximum(jnp.exp(0.5 * logvar), 1e-8)
    r2 = (STD_THETA / std) ** 2
    mu_ref[...] = mu
    r2_ref[...] = r2


def _mlp_call(x8, w1t8, b1c, w2t_bf16, b2c, w3t8, b3c):
    f32 = jnp.float32
    return pl.pallas_call(
        _mlp_body,
        out_shape=[jax.ShapeDtypeStruct((1, NC_PAD), f32),
                   jax.ShapeDtypeStruct((1, NC_PAD), f32)],
    )(x8, w1t8, b1c, w2t_bf16, b2c, w3t8, b3c)


_NC = 2
_NS = 16
_NW = _NC * _NS
_CHUNK = N // _NW
_LANES = 16


def _sc_gather_body(q_hbm, q2_hbm, dtab_hbm, ktab_hbm, mtab_hbm, rtab_hbm,
                    dout_hbm, kout_hbm, mout_hbm, rout_hbm,
                    idx_v, idx2_v, dtab_v, ktab_v, mtab_v, rtab_v,
                    dout_v, kout_v, mout_v, rout_v):
    wid = lax.axis_index("s") * _NC + lax.axis_index("c")
    base = wid * _CHUNK
    pltpu.sync_copy(q_hbm.at[pl.ds(base, _CHUNK)], idx_v)
    pltpu.sync_copy(q2_hbm.at[pl.ds(base, _CHUNK)], idx2_v)
    pltpu.sync_copy(dtab_hbm, dtab_v)
    pltpu.sync_copy(ktab_hbm, ktab_v)
    pltpu.sync_copy(mtab_hbm, mtab_v)
    pltpu.sync_copy(rtab_hbm, rtab_v)
    for j in range(_CHUNK // _LANES):
        sl = pl.ds(j * _LANES, _LANES)
        idx = idx_v[sl]
        idx2 = idx2_v[sl]
        dout_v[sl] = plsc.load_gather(dtab_v, [idx])
        kout_v[sl] = plsc.load_gather(ktab_v, [idx])
        mout_v[sl] = plsc.load_gather(mtab_v, [idx2])
        rout_v[sl] = plsc.load_gather(rtab_v, [idx2])
    pltpu.sync_copy(dout_v, dout_hbm.at[pl.ds(base, _CHUNK)])
    pltpu.sync_copy(kout_v, kout_hbm.at[pl.ds(base, _CHUNK)])
    pltpu.sync_copy(mout_v, mout_hbm.at[pl.ds(base, _CHUNK)])
    pltpu.sync_copy(rout_v, rout_hbm.at[pl.ds(base, _CHUNK)])


def _sc_gather(q_flat, q2_flat, dtab_pad, ktab_pad, mu_tab, r2_tab):
    mesh = plsc.VectorSubcoreMesh(core_axis_name="c", subcore_axis_name="s")
    f32 = jnp.float32
    call = pl.kernel(
        _sc_gather_body,
        mesh=mesh,
        compiler_params=pltpu.CompilerParams(needs_layout_passes=False),
        out_type=[jax.ShapeDtypeStruct((N,), f32) for _ in range(4)],
        scratch_types=[
            pltpu.VMEM((_CHUNK,), jnp.int32),
            pltpu.VMEM((_CHUNK,), jnp.int32),
            pltpu.VMEM((NQ_PAD,), f32),
            pltpu.VMEM((NQ_PAD,), f32),
            pltpu.VMEM((NC_PAD,), f32),
            pltpu.VMEM((NC_PAD,), f32),
            pltpu.VMEM((_CHUNK,), f32),
            pltpu.VMEM((_CHUNK,), f32),
            pltpu.VMEM((_CHUNK,), f32),
            pltpu.VMEM((_CHUNK,), f32),
        ],
    )
    return call(q_flat, q2_flat, dtab_pad, ktab_pad, mu_tab, r2_tab)


def _scan_body(mu_ref, r2_ref, diff_ref, disc_ref, logits_ref, last_ref,
               b_scr, c_scr):
    ones = jnp.ones((1, U), jnp.float32)
    zeros = jnp.zeros((1, U), jnp.float32)
    NT = S // TPB

    def bwd(t, carry):
        b_prev, c_prev = carry
        row0 = (NT - 1 - t) * TPB
        r2t = r2_ref[pl.ds(row0, TPB), :]
        mut = mu_ref[pl.ds(row0, TPB), :]
        bs, cs = [None] * TPB, [None] * TPB
        for j in range(TPB - 1, -1, -1):
            r2j = r2t[j:j + 1, :]
            b_prev = 1.0 / (2.0 + r2j - b_prev)
            c_prev = b_prev * (c_prev + r2j * mut[j:j + 1, :])
            bs[j] = b_prev
            cs[j] = c_prev
        b_scr[pl.ds(row0, TPB), :] = jnp.concatenate(bs, axis=0)
        c_scr[pl.ds(row0, TPB), :] = jnp.concatenate(cs, axis=0)
        return (b_prev, c_prev)

    lax.fori_loop(0, NT, bwd, (ones, zeros))

    def fwd(t, abil):
        row0 = t * TPB
        bt = b_scr[pl.ds(row0, TPB), :]
        ct = c_scr[pl.ds(row0, TPB), :]
        dt = diff_ref[pl.ds(row0, TPB), :]
        kt = disc_ref[pl.ds(row0, TPB), :]
        ls = [None] * TPB
        for j in range(TPB):
            abil = bt[j:j + 1, :] * abil + ct[j:j + 1, :]
            ls[j] = kt[j:j + 1, :] * (abil - dt[j:j + 1, :])
        logits_ref[pl.ds(row0, TPB), :] = jnp.concatenate(ls, axis=0)
        return abil

    a_last = lax.fori_loop(0, NT, fwd, zeros)
    last_ref[...] = a_last


def _scan_call(mu_t, r2_t, diff_t, disc_t):
    f32 = jnp.float32
    return pl.pallas_call(
        _scan_body,
        out_shape=[jax.ShapeDtypeStruct((S, U), f32),
                   jax.ShapeDtypeStruct((1, U), f32)],
        scratch_shapes=[pltpu.VMEM((S, U), f32), pltpu.VMEM((S, U), f32)],
    )(mu_t, r2_t, diff_t, disc_t)


def kernel(mask, q_id, kmap, resp, diff_mu_w, disc_mu_w, W1, b1, W2, b2, W3, b3):
    f32 = jnp.float32
    nq = diff_mu_w.shape[0]
    q_flat = q_id.T.reshape(N).astype(jnp.int32)
    resp_flat_i = resp.T.reshape(N).astype(jnp.int32)
    q2_flat = q_flat * 2 + resp_flat_i

    dtab = diff_mu_w[:, 0]
    ktab = disc_mu_w[:, 0]
    dtab_pad = jnp.zeros((NQ_PAD,), f32).at[:nq].set(dtab)
    ktab_pad = jnp.zeros((NQ_PAD,), f32).at[:nq].set(ktab)

    diff2 = jnp.zeros((NC_PAD,), f32).at[:2 * nq].set(jnp.repeat(dtab, 2))
    disc2 = jnp.zeros((NC_PAD,), f32).at[:2 * nq].set(jnp.repeat(ktab, 2))
    resp2 = jnp.zeros((NC_PAD,), f32).at[1:2 * nq:2].set(1.0)
    x8 = jnp.zeros((8, NC_PAD), f32)
    x8 = x8.at[0].set(diff2).at[1].set(disc2).at[2].set(resp2)

    w1t8 = jnp.zeros((H, 8), f32).at[:, :3].set(W1.T)
    w3t8 = jnp.zeros((8, H), f32).at[:2].set(W3.T)
    b3c = jnp.zeros((8, 1), f32).at[:2, 0].set(b3)

    mu_c, r2_c = _mlp_call(x8, w1t8, b1.reshape(H, 1),
                           W2.T.astype(jnp.bfloat16),
                           b2.reshape(H, 1), w3t8, b3c)

    diff_flat, disc_flat, mu_flat, r2_flat = _sc_gather(
        q_flat, q2_flat, dtab_pad, ktab_pad,
        mu_c.reshape(NC_PAD), r2_c.reshape(NC_PAD))

    logits_t, last = _scan_call(mu_flat.reshape(S, U), r2_flat.reshape(S, U),
                                diff_flat.reshape(S, U),
                                disc_flat.reshape(S, U))

    return logits_t.T, last.reshape(U, 1)


# speedup vs baseline: 2.3153x; 1.0108x over previous
import jax
import jax.numpy as jnp
from jax import lax
from jax.experimental import pallas as pl
from jax.experimental.pallas import tpu as pltpu
from jax.experimental.pallas import tpu_sc as plsc

H = 1024
S = 512
U = 64
N = S * U
NQ_PAD = 1024
NC_PAD = 2048
TPB = 8
STD_THETA = 1.0


_SQRT_HALF = 0.7071067811865476


def _gelu(x):
    return 0.5 * x * (1.0 + lax.erf(x * _SQRT_HALF))


_DN0 = (((0,), (0,)), ((), ()))


def _mlp_body(x8_ref, w1p_ref, b1_ref, w2bf_ref, b2_ref, w3p_ref, b3_ref,
              mu_ref, r2_ref):
    x = x8_ref[...]
    h = lax.dot_general(w1p_ref[...], x, _DN0,
                        preferred_element_type=jnp.float32)
    h = _gelu(h + b1_ref[...])
    h = lax.dot_general(w2bf_ref[...], h.astype(jnp.bfloat16), _DN0,
                        preferred_element_type=jnp.float32)
    h = _gelu(h + b2_ref[...])
    o = lax.dot_general(w3p_ref[...], h, _DN0,
                        preferred_element_type=jnp.float32)
    o = _gelu(o + b3_ref[...])
    mu = o[0:1, :]
    logvar = o[1:2, :]
    std = jnp.maximum(jnp.exp(0.5 * logvar), 1e-8)
    r2 = (STD_THETA / std) ** 2
    mu_ref[...] = mu
    r2_ref[...] = r2


def _mlp_call(x8, w1t8, b1c, w2t_bf16, b2c, w3t8, b3c):
    f32 = jnp.float32
    return pl.pallas_call(
        _mlp_body,
        out_shape=[jax.ShapeDtypeStruct((1, NC_PAD), f32),
                   jax.ShapeDtypeStruct((1, NC_PAD), f32)],
    )(x8, w1t8, b1c, w2t_bf16, b2c, w3t8, b3c)


_NC = 2
_NS = 16
_NW = _NC * _NS
_CHUNK = N // _NW
_LANES = 16


def _sc_gather_body(q_hbm, q2_hbm, dtab_hbm, ktab_hbm, mtab_hbm, rtab_hbm,
                    dout_hbm, kout_hbm, mout_hbm, rout_hbm,
                    idx_v, idx2_v, dtab_v, ktab_v, mtab_v, rtab_v,
                    dout_v, kout_v, mout_v, rout_v):
    wid = lax.axis_index("s") * _NC + lax.axis_index("c")
    base = wid * _CHUNK
    pltpu.sync_copy(q_hbm.at[pl.ds(base, _CHUNK)], idx_v)
    pltpu.sync_copy(q2_hbm.at[pl.ds(base, _CHUNK)], idx2_v)
    pltpu.sync_copy(dtab_hbm, dtab_v)
    pltpu.sync_copy(ktab_hbm, ktab_v)
    pltpu.sync_copy(mtab_hbm, mtab_v)
    pltpu.sync_copy(rtab_hbm, rtab_v)
    for j in range(_CHUNK // _LANES):
        sl = pl.ds(j * _LANES, _LANES)
        idx = idx_v[sl]
        idx2 = idx2_v[sl]
        dout_v[sl] = plsc.load_gather(dtab_v, [idx])
        kout_v[sl] = plsc.load_gather(ktab_v, [idx])
        mout_v[sl] = plsc.load_gather(mtab_v, [idx2])
        rout_v[sl] = plsc.load_gather(rtab_v, [idx2])
    pltpu.sync_copy(dout_v, dout_hbm.at[pl.ds(base, _CHUNK)])
    pltpu.sync_copy(kout_v, kout_hbm.at[pl.ds(base, _CHUNK)])
    pltpu.sync_copy(mout_v, mout_hbm.at[pl.ds(base, _CHUNK)])
    pltpu.sync_copy(rout_v, rout_hbm.at[pl.ds(base, _CHUNK)])


def _sc_gather(q_flat, q2_flat, dtab_pad, ktab_pad, mu_tab, r2_tab):
    mesh = plsc.VectorSubcoreMesh(core_axis_name="c", subcore_axis_name="s")
    f32 = jnp.float32
    call = pl.kernel(
        _sc_gather_body,
        mesh=mesh,
        compiler_params=pltpu.CompilerParams(needs_layout_passes=False),
        out_type=[jax.ShapeDtypeStruct((N,), f32) for _ in range(4)],
        scratch_types=[
            pltpu.VMEM((_CHUNK,), jnp.int32),
            pltpu.VMEM((_CHUNK,), jnp.int32),
            pltpu.VMEM((NQ_PAD,), f32),
            pltpu.VMEM((NQ_PAD,), f32),
            pltpu.VMEM((NC_PAD,), f32),
            pltpu.VMEM((NC_PAD,), f32),
            pltpu.VMEM((_CHUNK,), f32),
            pltpu.VMEM((_CHUNK,), f32),
            pltpu.VMEM((_CHUNK,), f32),
            pltpu.VMEM((_CHUNK,), f32),
        ],
    )
    return call(q_flat, q2_flat, dtab_pad, ktab_pad, mu_tab, r2_tab)


def _scan_body(mu_ref, r2_ref, diff_ref, disc_ref, logits_ref, last_ref,
               b_scr, c_scr):
    ones = jnp.ones((1, U), jnp.float32)
    zeros = jnp.zeros((1, U), jnp.float32)
    NT = S // TPB

    def bwd(t, carry):
        b_prev, c_prev = carry
        row0 = (NT - 1 - t) * TPB
        r2t = r2_ref[pl.ds(row0, TPB), :]
        mut = mu_ref[pl.ds(row0, TPB), :]
        bs, cs = [None] * TPB, [None] * TPB
        for j in range(TPB - 1, -1, -1):
            r2j = r2t[j:j + 1, :]
            b_prev = 1.0 / (2.0 + r2j - b_prev)
            c_prev = b_prev * (c_prev + r2j * mut[j:j + 1, :])
            bs[j] = b_prev
            cs[j] = c_prev
        b_scr[pl.ds(row0, TPB), :] = jnp.concatenate(bs, axis=0)
        c_scr[pl.ds(row0, TPB), :] = jnp.concatenate(cs, axis=0)
        return (b_prev, c_prev)

    lax.fori_loop(0, NT, bwd, (ones, zeros))

    def fwd(t, abil):
        row0 = t * TPB
        bt = b_scr[pl.ds(row0, TPB), :]
        ct = c_scr[pl.ds(row0, TPB), :]
        dt = diff_ref[pl.ds(row0, TPB), :]
        kt = disc_ref[pl.ds(row0, TPB), :]
        ls = [None] * TPB
        for j in range(TPB):
            abil = bt[j:j + 1, :] * abil + ct[j:j + 1, :]
            ls[j] = kt[j:j + 1, :] * (abil - dt[j:j + 1, :])
        logits_ref[pl.ds(row0, TPB), :] = jnp.concatenate(ls, axis=0)
        return abil

    a_last = lax.fori_loop(0, NT, fwd, zeros)
    last_ref[...] = a_last


def _scan_call(mu_t, r2_t, diff_t, disc_t):
    f32 = jnp.float32
    return pl.pallas_call(
        _scan_body,
        out_shape=[jax.ShapeDtypeStruct((S, U), f32),
                   jax.ShapeDtypeStruct((1, U), f32)],
        scratch_shapes=[pltpu.VMEM((S, U), f32), pltpu.VMEM((S, U), f32)],
    )(mu_t, r2_t, diff_t, disc_t)


def kernel(mask, q_id, kmap, resp, diff_mu_w, disc_mu_w, W1, b1, W2, b2, W3, b3):
    f32 = jnp.float32
    nq = diff_mu_w.shape[0]
    q_flat = q_id.T.reshape(N).astype(jnp.int32)
    resp_flat_i = resp.T.reshape(N).astype(jnp.int32)
    q2_flat = q_flat * 2 + resp_flat_i

    dtab = diff_mu_w[:, 0]
    ktab = disc_mu_w[:, 0]
    dtab_pad = jnp.zeros((NQ_PAD,), f32).at[:nq].set(dtab)
    ktab_pad = jnp.zeros((NQ_PAD,), f32).at[:nq].set(ktab)

    diff2 = jnp.zeros((NC_PAD,), f32).at[:2 * nq].set(jnp.repeat(dtab, 2))
    disc2 = jnp.zeros((NC_PAD,), f32).at[:2 * nq].set(jnp.repeat(ktab, 2))
    resp2 = jnp.zeros((NC_PAD,), f32).at[1:2 * nq:2].set(1.0)
    x8 = jnp.zeros((8, NC_PAD), f32)
    x8 = x8.at[0].set(diff2).at[1].set(disc2).at[2].set(resp2)

    w1p = jnp.zeros((8, H), f32).at[:3].set(W1)
    w3p = jnp.zeros((H, 8), f32).at[:, :2].set(W3)
    b3c = jnp.zeros((8, 1), f32).at[:2, 0].set(b3)

    mu_c, r2_c = _mlp_call(x8, w1p, b1.reshape(H, 1),
                           W2.astype(jnp.bfloat16),
                           b2.reshape(H, 1), w3p, b3c)

    diff_flat, disc_flat, mu_flat, r2_flat = _sc_gather(
        q_flat, q2_flat, dtab_pad, ktab_pad,
        mu_c.reshape(NC_PAD), r2_c.reshape(NC_PAD))

    logits_t, last = _scan_call(mu_flat.reshape(S, U), r2_flat.reshape(S, U),
                                diff_flat.reshape(S, U),
                                disc_flat.reshape(S, U))

    return logits_t.T, last.reshape(U, 1)
